# baseline (speedup 1.0000x reference)
import jax
import jax.numpy as jnp
from jax.experimental import pallas as pl

N = 10000
E = 160000
D = 256
G = 16


def _final_linear_body(pooled_ref, wl_ref, bl_ref, out_ref):
    out_ref[...] = pooled_ref[...] @ wl_ref[...] + bl_ref[0]


def kernel(x, edge_index, edge_attr, batch, W_c0, W_c1, b_c, W_g, a_src, a_dst, b_g, W_l, b_l):
    row = edge_index[0]
    col = edge_index[1]
    deg = jax.ops.segment_sum(edge_attr, row, num_segments=N)
    dinv = jnp.where(deg > 0, 1.0 / jnp.sqrt(deg), 0.0)
    norm = -dinv[row] * edge_attr * dinv[col]
    msg = norm[:, None] * x[row]
    Tx1 = jax.ops.segment_sum(msg, col, num_segments=N)
    h = x @ W_c0 + Tx1 @ W_c1 + b_c
    h = jax.nn.relu(h)
    hp = h @ W_g
    loop = jnp.arange(N, dtype=row.dtype)
    r2 = jnp.concatenate([row, loop])
    c2 = jnp.concatenate([col, loop])
    asrc = hp @ a_src
    adst = hp @ a_dst
    e = asrc[r2] + adst[c2]
    e = jnp.where(e > 0, e, 0.2 * e)
    emax = jax.ops.segment_max(e, c2, num_segments=N)
    emax = jnp.where(jnp.isfinite(emax), emax, 0.0)
    ex = jnp.exp(e - emax[c2])
    esum = jax.ops.segment_sum(ex, c2, num_segments=N)
    alpha = ex / (esum[c2] + 1e-16)
    out = jax.ops.segment_sum(alpha[:, None] * hp[r2], c2, num_segments=N) + b_g
    out = jax.nn.relu(out)
    summed = jax.ops.segment_sum(out, batch, num_segments=G)
    cnt = jax.ops.segment_sum(jnp.ones((N,), jnp.float32), batch, num_segments=G)
    pooled = summed / jnp.maximum(cnt, 1.0)[:, None]
    return pl.pallas_call(
        _final_linear_body,
        out_shape=jax.ShapeDtypeStruct((G, 1), jnp.float32),
    )(pooled, W_l, b_l)


# speedup vs baseline: 7.4734x; 7.4734x over previous
import functools

import jax
import jax.numpy as jnp
from jax import lax
from jax.experimental import pallas as pl
from jax.experimental.pallas import tpu as pltpu
from jax.experimental.pallas import tpu_sc as plsc

N = 10000
E = 160000
D = 256
G = 16

NP = 10240
EP = 163840
EAC = EP // 32
GAC = EAC // 16
CH = 128
EPT = EP // 16
NCH = EPT // CH
ROWS_PER_TILE = NP // 16

_MESH = dict(core_axis_name="c", subcore_axis_name="s", num_cores=2,
             num_subcores=16)

f32 = jnp.float32
i32 = jnp.int32


def _wid():
    return lax.axis_index("s") * 2 + lax.axis_index("c")


def _sc_deg_body(row_h, ea_h, degp_h, idx_v, val_v, acc_v):
    wid = _wid()
    base = wid * EAC
    pltpu.sync_copy(row_h.at[pl.ds(base, EAC)], idx_v)
    pltpu.sync_copy(ea_h.at[pl.ds(base, EAC)], val_v)

    def _zero(i, _):
        acc_v[pl.ds(i * 16, 16)] = jnp.zeros((16,), f32)
        return _
    lax.fori_loop(0, NP // 16, _zero, None)

    def _grp(g, _):
        sl = pl.ds(g * 16, 16)
        plsc.addupdate_scatter(acc_v, [idx_v[sl]], val_v[sl])
        return _
    lax.fori_loop(0, GAC, _grp, None)
    pltpu.sync_copy(acc_v, degp_h.at[pl.ds(wid * NP, NP)])


def _sc_deg(rowp, eap):
    return pl.kernel(
        _sc_deg_body,
        out_type=jax.ShapeDtypeStruct((32 * NP,), f32),
        mesh=plsc.VectorSubcoreMesh(**_MESH),
        compiler_params=pltpu.CompilerParams(needs_layout_passes=False),
        scratch_types=[
            pltpu.VMEM((EAC,), i32),
            pltpu.VMEM((EAC,), f32),
            pltpu.VMEM((NP,), f32),
        ],
    )(rowp, eap)


def _sc_gat_body(row_h, col_h, asrc_h, adst_h, bcap_h, elsh_h,
                 idxr_v, idxc_v, asrc_v, adst_v, bcap_v, w_v):
    wid = _wid()
    base = wid * EAC
    pltpu.sync_copy(row_h.at[pl.ds(base, EAC)], idxr_v)
    pltpu.sync_copy(col_h.at[pl.ds(base, EAC)], idxc_v)
    pltpu.sync_copy(asrc_h, asrc_v)
    pltpu.sync_copy(adst_h, adst_v)
    pltpu.sync_copy(bcap_h, bcap_v)

    lanes = lax.iota(i32, 16)

    def _grp(g, _):
        sl = pl.ds(g * 16, 16)
        ir = idxr_v[sl]
        ic = idxc_v[sl]
        el = (plsc.load_gather(asrc_v, [ir])
              + plsc.load_gather(adst_v, [ic]))
        el = jnp.where(el > 0, el, 0.2 * el)
        el = el - plsc.load_gather(bcap_v, [ic])
        eid = base + g * 16 + lanes
        w_v[sl] = jnp.where(eid < E, el, -1e4)
        return _
    lax.fori_loop(0, GAC, _grp, None)
    pltpu.sync_copy(w_v, elsh_h.at[pl.ds(base, EAC)])


def _sc_gat(rowp, colp, asrc1, adst1, bcap1):
    return pl.kernel(
        _sc_gat_body,
        out_type=jax.ShapeDtypeStruct((EP,), f32),
        mesh=plsc.VectorSubcoreMesh(**_MESH),
        compiler_params=pltpu.CompilerParams(needs_layout_passes=False),
        scratch_types=[
            pltpu.VMEM((EAC,), i32),
            pltpu.VMEM((EAC,), i32),
            pltpu.VMEM((NP,), f32),
            pltpu.VMEM((NP,), f32),
            pltpu.VMEM((NP,), f32),
            pltpu.VMEM((EAC,), f32),
        ],
    )(rowp, colp, asrc1, adst1, bcap1)


def _t2c_body(el_ref, w_ref):
    w_ref[...] = jnp.exp(el_ref[...])


def _t2c(elsh2):
    return pl.pallas_call(
        _t2c_body,
        out_shape=jax.ShapeDtypeStruct((EP // 128, 128), f32),
    )(elsh2)


def _sc_agg_body(tab_h, row2_h, col_h, w_h, out0_h, out1_h,
                 row2_v, col_v, w_v, rows_v, gidx_v, cidx_v, acc_sh, sem):
    cid = lax.axis_index("c")
    sid = lax.axis_index("s")
    ebase = sid * EPT
    pltpu.sync_copy(row2_h.at[pl.ds(ebase, EPT)], row2_v)
    pltpu.sync_copy(col_h.at[pl.ds(ebase, EPT)], col_v)
    pltpu.sync_copy(w_h.at[pl.ds(ebase, EPT)], w_v)

    for r in range(CH):
        for j in range(8):
            rows_v[r, pl.ds(j * 16, 16)] = jnp.zeros((16,), f32)
    for i in range(ROWS_PER_TILE // CH):
        pltpu.sync_copy(rows_v, acc_sh.at[pl.ds(sid * ROWS_PER_TILE + i * CH, CH)])
    plsc.subcore_barrier()

    def _chunk(c, _):
        for j in range(CH // 16):
            sl = pl.ds(j * 16, 16)
            esl = pl.ds(c * CH + j * 16, 16)
            gidx_v[sl] = row2_v[esl] + cid
            cidx_v[sl] = col_v[esl]
        pltpu.async_copy(tab_h.at[gidx_v], rows_v, sem).wait()
        for r in range(CH):
            wspl = plsc.load_gather(w_v, [jnp.full((16,), c * CH + r, i32)])
            for j in range(8):
                sl = (r, pl.ds(j * 16, 16))
                rows_v[sl] = rows_v[sl] * wspl
        pltpu.sync_copy(rows_v, acc_sh.at[cidx_v], add=True)
        return _
    lax.fori_loop(0, NCH, _chunk, None)
    plsc.subcore_barrier()

    @pl.when(cid == 0)
    def _():
        pltpu.sync_copy(acc_sh.at[pl.ds(sid * ROWS_PER_TILE, ROWS_PER_TILE)],
                        out0_h.at[pl.ds(sid * ROWS_PER_TILE, ROWS_PER_TILE)])

    @pl.when(cid == 1)
    def _():
        pltpu.sync_copy(acc_sh.at[pl.ds(sid * ROWS_PER_TILE, ROWS_PER_TILE)],
                        out1_h.at[pl.ds(sid * ROWS_PER_TILE, ROWS_PER_TILE)])


def _sc_agg(tab2, row2, colp, w):
    return pl.kernel(
        _sc_agg_body,
        out_type=[
            jax.ShapeDtypeStruct((NP, 128), f32),
            jax.ShapeDtypeStruct((NP, 128), f32),
        ],
        mesh=plsc.VectorSubcoreMesh(**_MESH),
        compiler_params=pltpu.CompilerParams(needs_layout_passes=False),
        scratch_types=[
            pltpu.VMEM((EPT,), i32),
            pltpu.VMEM((EPT,), i32),
            pltpu.VMEM((EPT,), f32),
            pltpu.VMEM((CH, 128), f32),
            pltpu.VMEM((CH,), i32),
            pltpu.VMEM((CH,), i32),
            pltpu.VMEM_SHARED((NP, 128), f32),
            pltpu.SemaphoreType.DMA,
        ],
    )(tab2, row2, colp, w)


BR = 1024
NB = NP // BR


def _t1_body(x_ref, wc0_ref, wc1_ref, degp_ref, ones_ref,
             x0w_ref, y1_ref, dinv_ref):
    deg = lax.dot_general(degp_ref[...], ones_ref[...],
                          (((0,), (0,)), ((), ())),
                          preferred_element_type=f32,
                          precision=lax.Precision.HIGHEST)
    dinv = jnp.where(deg > 0, lax.rsqrt(jnp.maximum(deg, 1e-30)), 0.0)
    xb = x_ref[...]
    x0w_ref[...] = jnp.dot(xb, wc0_ref[...], preferred_element_type=f32, precision=lax.Precision.HIGHEST)
    y1_ref[...] = dinv * jnp.dot(xb, wc1_ref[...], preferred_element_type=f32, precision=lax.Precision.HIGHEST)
    dinv_ref[...] = dinv


def _t1(xp, W_c0, W_c1, degp):
    ones = jnp.ones((32, 1), f32)
    return pl.pallas_call(
        _t1_body,
        grid=(NB,),
        in_specs=[
            pl.BlockSpec((BR, D), lambda i: (i, 0)),
            pl.BlockSpec((D, D), lambda i: (0, 0)),
            pl.BlockSpec((D, D), lambda i: (0, 0)),
            pl.BlockSpec((32, BR), lambda i: (0, i)),
            pl.BlockSpec((32, 1), lambda i: (0, 0)),
        ],
        out_specs=[
            pl.BlockSpec((BR, D), lambda i: (i, 0)),
            pl.BlockSpec((BR, D), lambda i: (i, 0)),
            pl.BlockSpec((BR, 1), lambda i: (i, 0)),
        ],
        out_shape=[
            jax.ShapeDtypeStruct((NP, D), f32),
            jax.ShapeDtypeStruct((NP, D), f32),
            jax.ShapeDtypeStruct((NP, 1), f32),
        ],
    )(xp, W_c0, W_c1, degp, ones)


def _t2_body(x0w_ref, s0_ref, s1_ref, dinv_ref, bc_ref, wg_ref,
             av_ref, ad_ref, hp_ref, asrc_ref, adst_ref, mpart_ref):
    i = pl.program_id(0)
    s = jnp.concatenate([s0_ref[...], s1_ref[...]], axis=1)
    h = x0w_ref[...] + dinv_ref[...] * s + bc_ref[...]
    h = jnp.maximum(h, 0.0)
    hp = jnp.dot(h, wg_ref[...], preferred_element_type=f32, precision=lax.Precision.HIGHEST)
    hp_ref[...] = hp
    asrc = jnp.dot(hp, av_ref[...], preferred_element_type=f32, precision=lax.Precision.HIGHEST)
    adst = jnp.dot(hp, ad_ref[...], preferred_element_type=f32, precision=lax.Precision.HIGHEST)
    rid = i * BR + lax.broadcasted_iota(i32, (BR, 1), 0)
    asrc = jnp.where(rid < N, asrc, -1e30)
    adst = jnp.where(rid < N, adst, -1e30)
    asrc_ref[...] = asrc
    adst_ref[...] = adst
    mpart_ref[...] = jnp.full((BR, 1), jnp.max(asrc), f32)


def _t2(x0w, s0, s1, dinv, bc2, W_g, av2, ad2):
    return pl.pallas_call(
        _t2_body,
        grid=(NB,),
        in_specs=[
            pl.BlockSpec((BR, D), lambda i: (i, 0)),
            pl.BlockSpec((BR, 128), lambda i: (i, 0)),
            pl.BlockSpec((BR, 128), lambda i: (i, 0)),
            pl.BlockSpec((BR, 1), lambda i: (i, 0)),
            pl.BlockSpec((1, D), lambda i: (0, 0)),
            pl.BlockSpec((D, D), lambda i: (0, 0)),
            pl.BlockSpec((D, 1), lambda i: (0, 0)),
            pl.BlockSpec((D, 1), lambda i: (0, 0)),
        ],
        out_specs=[
            pl.BlockSpec((BR, D), lambda i: (i, 0)),
            pl.BlockSpec((BR, 1), lambda i: (i, 0)),
            pl.BlockSpec((BR, 1), lambda i: (i, 0)),
            pl.BlockSpec((BR, 1), lambda i: (i, 0)),
        ],
        out_shape=[
            jax.ShapeDtypeStruct((NP, D), f32),
            jax.ShapeDtypeStruct((NP, 1), f32),
            jax.ShapeDtypeStruct((NP, 1), f32),
            jax.ShapeDtypeStruct((NP, 1), f32),
        ],
    )(x0w, s0, s1, dinv, bc2, W_g, av2, ad2)


def _t2b_body(asrc_ref, adst_ref, mpart_ref, bcap_ref, selfw_ref):
    m = jnp.max(mpart_ref[...])
    adst = adst_ref[...]
    asrc = asrc_ref[...]
    v = m + adst
    bcap = jnp.where(v > 0, v, 0.2 * v)
    e = asrc + adst
    e = jnp.where(e > 0, e, 0.2 * e)
    bcap_ref[...] = bcap
    selfw_ref[...] = jnp.exp(e - bcap)


def _t2b(asrcP, adstP, mpart):
    return pl.pallas_call(
        _t2b_body,
        out_shape=[
            jax.ShapeDtypeStruct((NP, 1), f32),
            jax.ShapeDtypeStruct((NP, 1), f32),
        ],
    )(asrcP, adstP, mpart)


def _t3_body(sn0_ref, sn1_ref, hp_ref, selfw_ref, esump_ref, ones32_ref,
             batch_ref, bg_ref, wl_ref, bl_ref, out_ref):
    esum = lax.dot_general(esump_ref[...], ones32_ref[...],
                           (((0,), (0,)), ((), ())),
                           preferred_element_type=f32, precision=lax.Precision.HIGHEST)
    selfw = selfw_ref[...]
    hp = hp_ref[...]
    num = jnp.concatenate([sn0_ref[...], sn1_ref[...]], axis=1) + selfw * hp
    den = esum + selfw + 1e-16
    out = jnp.maximum(num / den + bg_ref[...], 0.0)
    oh = (batch_ref[...] == lax.broadcasted_iota(i32, (1, G), 1)).astype(f32)
    ps = lax.dot_general(oh, out, (((0,), (0,)), ((), ())),
                         preferred_element_type=f32, precision=lax.Precision.HIGHEST)
    cnt = lax.dot_general(oh, jnp.ones((NP, 1), f32), (((0,), (0,)), ((), ())),
                          preferred_element_type=f32, precision=lax.Precision.HIGHEST)
    pooled = ps / jnp.maximum(cnt, 1.0)
    out_ref[...] = (jnp.dot(pooled, wl_ref[...], preferred_element_type=f32, precision=lax.Precision.HIGHEST)
                    + bl_ref[...])


def _t3(sn0, sn1, hp, selfwP, esump, batchP, bg2, W_l, bl2):
    ones32 = jnp.ones((32, 1), f32)
    return pl.pallas_call(
        _t3_body,
        out_shape=jax.ShapeDtypeStruct((G, 1), f32),
    )(sn0, sn1, hp, selfwP, esump, ones32, batchP, bg2, W_l, bl2)


def kernel(x, edge_index, edge_attr, batch, W_c0, W_c1, b_c, W_g, a_src,
           a_dst, b_g, W_l, b_l):
    row = edge_index[0]
    col = edge_index[1]
    pad = EP - E
    rowp = jnp.concatenate([row, jnp.zeros((pad,), i32)])
    colp = jnp.concatenate([col, jnp.zeros((pad,), i32)])
    eap = jnp.concatenate([edge_attr, jnp.zeros((pad,), f32)])
    row2 = 2 * rowp
    xp = jnp.pad(x, ((0, NP - N), (0, 0)))
    batchP = jnp.concatenate([batch, jnp.full((NP - N,), G, i32)]).reshape(NP, 1)
    bc2 = b_c.reshape(1, D)
    bg2 = b_g.reshape(1, D)
    av2 = a_src.reshape(D, 1)
    ad2 = a_dst.reshape(D, 1)
    bl2 = b_l.reshape(1, 1)

    degp = _sc_deg(rowp, eap)
    x0w, y1, dinv = _t1(xp, W_c0, W_c1, degp.reshape(32, NP))
    s0, s1 = _sc_agg(y1.reshape(2 * NP, 128), row2, colp, -eap)

    hp, asrcP, adstP, mpart = _t2(x0w, s0, s1, dinv, bc2, W_g, av2, ad2)
    bcapP, selfwP = _t2b(asrcP, adstP, mpart)
    elsh = _sc_gat(rowp, colp, asrcP.reshape(NP), adstP.reshape(NP),
                   bcapP.reshape(NP))
    wE = _t2c(elsh.reshape(EP // 128, 128)).reshape(EP)
    esump = _sc_deg(colp, wE)
    sn0, sn1 = _sc_agg(hp.reshape(2 * NP, 128), row2, colp, wE)

    return _t3(sn0, sn1, hp, selfwP, esump.reshape(32, NP), batchP, bg2, W_l,
               bl2)


# speedup vs baseline: 10.2790x; 1.3754x over previous
import functools

import jax
import jax.numpy as jnp
from jax import lax
from jax.experimental import pallas as pl
from jax.experimental.pallas import tpu as pltpu
from jax.experimental.pallas import tpu_sc as plsc

N = 10000
E = 160000
D = 256
G = 16

NP = 10240
EP = 163840
EAC = EP // 32
GAC = EAC // 16
CH = 128
EPT = EP // 16
NSEG = 2
SEGE = EPT // NSEG
SEGC = SEGE // CH
ROWS_PER_TILE = NP // 16

_MESH = dict(core_axis_name="c", subcore_axis_name="s", num_cores=2,
             num_subcores=16)

f32 = jnp.float32
i32 = jnp.int32


def _recip(b):
    t0 = 1.0 / b
    t1 = t0 + t0 * (1.0 - t0 * b)
    return t1 + t1 * (1.0 - t1 * b)


def _wid():
    return lax.axis_index("s") * 2 + lax.axis_index("c")


def _sc_deg_body(row_h, ea_h, degp_h, idx_v, val_v, acc_v):
    wid = _wid()
    base = wid * EAC
    pltpu.sync_copy(row_h.at[pl.ds(base, EAC)], idx_v)
    pltpu.sync_copy(ea_h.at[pl.ds(base, EAC)], val_v)

    def _zero(i, _):
        acc_v[pl.ds(i * 16, 16)] = jnp.zeros((16,), f32)
        return _
    lax.fori_loop(0, NP // 16, _zero, None)

    def _grp(g, _):
        sl = pl.ds(g * 16, 16)
        plsc.addupdate_scatter(acc_v, [idx_v[sl]], val_v[sl])
        return _
    lax.fori_loop(0, GAC, _grp, None)
    pltpu.sync_copy(acc_v, degp_h.at[pl.ds(wid * NP, NP)])


def _sc_deg(rowp, eap):
    return pl.kernel(
        _sc_deg_body,
        out_type=jax.ShapeDtypeStruct((32 * NP,), f32),
        mesh=plsc.VectorSubcoreMesh(**_MESH),
        compiler_params=pltpu.CompilerParams(needs_layout_passes=False),
        scratch_types=[
            pltpu.VMEM((EAC,), i32),
            pltpu.VMEM((EAC,), f32),
            pltpu.VMEM((NP,), f32),
        ],
    )(rowp, eap)


def _sc_gat_body(row_h, col_h, asrc_h, adst_h, bcap_h, elsh_h,
                 idxr_v, idxc_v, asrc_v, adst_v, bcap_v, w_v):
    wid = _wid()
    base = wid * EAC
    pltpu.sync_copy(row_h.at[pl.ds(base, EAC)], idxr_v)
    pltpu.sync_copy(col_h.at[pl.ds(base, EAC)], idxc_v)
    pltpu.sync_copy(asrc_h, asrc_v)
    pltpu.sync_copy(adst_h, adst_v)
    pltpu.sync_copy(bcap_h, bcap_v)

    lanes = lax.iota(i32, 16)

    def _grp(g, _):
        sl = pl.ds(g * 16, 16)
        ir = idxr_v[sl]
        ic = idxc_v[sl]
        el = (plsc.load_gather(asrc_v, [ir])
              + plsc.load_gather(adst_v, [ic]))
        el = jnp.where(el > 0, el, 0.2 * el)
        el = el - plsc.load_gather(bcap_v, [ic])
        eid = base + g * 16 + lanes
        w_v[sl] = jnp.where(eid < E, el, -1e4)
        return _
    lax.fori_loop(0, GAC, _grp, None)
    pltpu.sync_copy(w_v, elsh_h.at[pl.ds(base, EAC)])


def _sc_gat(rowp, colp, asrc1, adst1, bcap1):
    return pl.kernel(
        _sc_gat_body,
        out_type=jax.ShapeDtypeStruct((EP,), f32),
        mesh=plsc.VectorSubcoreMesh(**_MESH),
        compiler_params=pltpu.CompilerParams(needs_layout_passes=False),
        scratch_types=[
            pltpu.VMEM((EAC,), i32),
            pltpu.VMEM((EAC,), i32),
            pltpu.VMEM((NP,), f32),
            pltpu.VMEM((NP,), f32),
            pltpu.VMEM((NP,), f32),
            pltpu.VMEM((EAC,), f32),
        ],
    )(rowp, colp, asrc1, adst1, bcap1)


def _t2c_body(el_ref, w_ref):
    w_ref[...] = jnp.exp(el_ref[...])


def _t2c(elsh2):
    return pl.pallas_call(
        _t2c_body,
        out_shape=jax.ShapeDtypeStruct((EP // 128, 128), f32),
    )(elsh2)


def _sc_agg_body(tab_h, row2_h, col_h, w_h, out0_h, out1_h,
                 row2_v, col_v, w_v, rows_v, rows2_v, gidx_v, gidx2_v,
                 cidx_v, cidx2_v, acc_sh, sem, sem2):
    cid = lax.axis_index("c")
    sid = lax.axis_index("s")

    def _zrow(r, _):
        for j in range(8):
            rows_v[r, pl.ds(j * 16, 16)] = jnp.zeros((16,), f32)
        return _
    lax.fori_loop(0, CH, _zrow, None)
    for i in range(ROWS_PER_TILE // CH):
        pltpu.sync_copy(rows_v, acc_sh.at[pl.ds(sid * ROWS_PER_TILE + i * CH, CH)])
    plsc.subcore_barrier()

    def _build(c, gidx, cidx):
        for j in range(CH // 16):
            sl = pl.ds(j * 16, 16)
            esl = pl.ds(c * CH + j * 16, 16)
            gidx[sl] = row2_v[esl] + cid
            cidx[sl] = col_v[esl]

    def _scale(cbase, rows):
        def _rg(rg, _):
            for r8 in range(8):
                r = rg * 8 + r8
                wspl = plsc.load_gather(w_v, [jnp.full((16,), cbase + r, i32)])
                for j in range(8):
                    sl = (r, pl.ds(j * 16, 16))
                    rows[sl] = rows[sl] * wspl
            return _
        lax.fori_loop(0, CH // 8, _rg, None)

    def _seg(seg, _):
        ebase = sid * EPT + seg * SEGE
        pltpu.sync_copy(row2_h.at[pl.ds(ebase, SEGE)], row2_v)
        pltpu.sync_copy(col_h.at[pl.ds(ebase, SEGE)], col_v)
        pltpu.sync_copy(w_h.at[pl.ds(ebase, SEGE)], w_v)

        _build(0, gidx_v, cidx_v)
        pltpu.make_async_copy(tab_h.at[gidx_v], rows_v, sem).start()

        def _pair(t, _):
            c0 = 2 * t
            _build(c0 + 1, gidx2_v, cidx2_v)
            pltpu.make_async_copy(tab_h.at[gidx2_v], rows2_v, sem2).start()
            pltpu.make_async_copy(tab_h.at[gidx_v], rows_v, sem).wait()
            _scale(c0 * CH, rows_v)
            pltpu.sync_copy(rows_v, acc_sh.at[cidx_v], add=True)

            @pl.when(c0 + 2 < SEGC)
            def _():
                _build(c0 + 2, gidx_v, cidx_v)
                pltpu.make_async_copy(tab_h.at[gidx_v], rows_v, sem).start()
            pltpu.make_async_copy(tab_h.at[gidx2_v], rows2_v, sem2).wait()
            _scale((c0 + 1) * CH, rows2_v)
            pltpu.sync_copy(rows2_v, acc_sh.at[cidx2_v], add=True)
            return _
        lax.fori_loop(0, SEGC // 2, _pair, None)
        return _
    lax.fori_loop(0, NSEG, _seg, None)
    plsc.subcore_barrier()

    @pl.when(cid == 0)
    def _():
        pltpu.sync_copy(acc_sh.at[pl.ds(sid * ROWS_PER_TILE, ROWS_PER_TILE)],
                        out0_h.at[pl.ds(sid * ROWS_PER_TILE, ROWS_PER_TILE)])

    @pl.when(cid == 1)
    def _():
        pltpu.sync_copy(acc_sh.at[pl.ds(sid * ROWS_PER_TILE, ROWS_PER_TILE)],
                        out1_h.at[pl.ds(sid * ROWS_PER_TILE, ROWS_PER_TILE)])


def _sc_agg(tab2, row2, colp, w):
    return pl.kernel(
        _sc_agg_body,
        out_type=[
            jax.ShapeDtypeStruct((NP, 128), f32),
            jax.ShapeDtypeStruct((NP, 128), f32),
        ],
        mesh=plsc.VectorSubcoreMesh(**_MESH),
        compiler_params=pltpu.CompilerParams(needs_layout_passes=False),
        scratch_types=[
            pltpu.VMEM((SEGE,), i32),
            pltpu.VMEM((SEGE,), i32),
            pltpu.VMEM((SEGE,), f32),
            pltpu.VMEM((CH, 128), f32),
            pltpu.VMEM((CH, 128), f32),
            pltpu.VMEM((CH,), i32),
            pltpu.VMEM((CH,), i32),
            pltpu.VMEM((CH,), i32),
            pltpu.VMEM((CH,), i32),
            pltpu.VMEM_SHARED((NP, 128), f32),
            pltpu.SemaphoreType.DMA,
            pltpu.SemaphoreType.DMA,
        ],
    )(tab2, row2, colp, w)


def _sc_max_body(row_h, col_h, asrc_h, msrc_h,
                 idxr_v, idxc_v, asrc_v, acc_v, comb_v, shared_sh):
    cid = lax.axis_index("c")
    sid = lax.axis_index("s")
    wid = sid * 2 + cid
    base = wid * EAC
    pltpu.sync_copy(row_h.at[pl.ds(base, EAC)], idxr_v)
    pltpu.sync_copy(col_h.at[pl.ds(base, EAC)], idxc_v)
    pltpu.sync_copy(asrc_h, asrc_v)

    def _init(i, _):
        acc_v[pl.ds(i * 16, 16)] = jnp.full((16,), -1e30, f32)
        return _
    lax.fori_loop(0, NP // 16, _init, None)

    lanes = lax.iota(i32, 16)

    def _grp(g, _):
        sl = pl.ds(g * 16, 16)
        ir = idxr_v[sl]
        ic = idxc_v[sl]
        val = plsc.load_gather(asrc_v, [ir])
        eid = base + g * 16 + lanes
        val = jnp.where(eid < E, val, -1e30)
        for l in range(16):
            cur = plsc.load_gather(acc_v, [ic])
            plsc.store_scatter(acc_v, [ic], jnp.maximum(cur, val),
                               mask=lanes == l)
        return _
    lax.fori_loop(0, GAC, _grp, None)

    pltpu.sync_copy(acc_v, shared_sh.at[sid])
    plsc.subcore_barrier()
    nbase = sid * ROWS_PER_TILE
    pltpu.sync_copy(shared_sh.at[:, pl.ds(nbase, ROWS_PER_TILE)], comb_v)

    def _comb(i, _):
        sl = pl.ds(i * 16, 16)
        m = comb_v[0, sl]
        for k in range(1, 16):
            m = jnp.maximum(m, comb_v[k, sl])
        acc_v[sl] = m
        return _
    lax.fori_loop(0, ROWS_PER_TILE // 16, _comb, None)
    pltpu.sync_copy(acc_v.at[pl.ds(0, ROWS_PER_TILE)],
                    msrc_h.at[pl.ds(cid * NP + nbase, ROWS_PER_TILE)])


def _sc_max(rowp, colp, asrc1):
    return pl.kernel(
        _sc_max_body,
        out_type=jax.ShapeDtypeStruct((2 * NP,), f32),
        mesh=plsc.VectorSubcoreMesh(**_MESH),
        compiler_params=pltpu.CompilerParams(needs_layout_passes=False),
        scratch_types=[
            pltpu.VMEM((EAC,), i32),
            pltpu.VMEM((EAC,), i32),
            pltpu.VMEM((NP,), f32),
            pltpu.VMEM((NP,), f32),
            pltpu.VMEM((16, ROWS_PER_TILE), f32),
            pltpu.VMEM_SHARED((16, NP), f32),
        ],
    )(rowp, colp, asrc1)


BR = 1024
NB = NP // BR


def _t1_body(x_ref, wc0_ref, wc1_ref, degp_ref, ones_ref,
             x0w_ref, y1_ref, dinv_ref):
    deg = lax.dot_general(degp_ref[...], ones_ref[...],
                          (((0,), (0,)), ((), ())),
                          preferred_element_type=f32,
                          precision=lax.Precision.HIGHEST)
    y = lax.rsqrt(jnp.maximum(deg, 1e-30))
    y = y * (1.5 - 0.5 * deg * y * y)
    y = y * (1.5 - 0.5 * deg * y * y)
    dinv = jnp.where(deg > 0, y, 0.0)
    xb = x_ref[...]
    x0w_ref[...] = jnp.dot(xb, wc0_ref[...], preferred_element_type=f32)
    y1_ref[...] = dinv * jnp.dot(xb, wc1_ref[...], preferred_element_type=f32)
    dinv_ref[...] = dinv


def _t1(xp, W_c0, W_c1, degp):
    ones = jnp.ones((32, 1), f32)
    return pl.pallas_call(
        _t1_body,
        grid=(NB,),
        in_specs=[
            pl.BlockSpec((BR, D), lambda i: (i, 0)),
            pl.BlockSpec((D, D), lambda i: (0, 0)),
            pl.BlockSpec((D, D), lambda i: (0, 0)),
            pl.BlockSpec((32, BR), lambda i: (0, i)),
            pl.BlockSpec((32, 1), lambda i: (0, 0)),
        ],
        out_specs=[
            pl.BlockSpec((BR, D), lambda i: (i, 0)),
            pl.BlockSpec((BR, D), lambda i: (i, 0)),
            pl.BlockSpec((BR, 1), lambda i: (i, 0)),
        ],
        out_shape=[
            jax.ShapeDtypeStruct((NP, D), f32),
            jax.ShapeDtypeStruct((NP, D), f32),
            jax.ShapeDtypeStruct((NP, 1), f32),
        ],
    )(xp, W_c0, W_c1, degp, ones)


def _t2_body(x0w_ref, s0_ref, s1_ref, dinv_ref, bc_ref, wg_ref,
             av_ref, ad_ref, hp_ref, asrc_ref, adst_ref, mpart_ref):
    i = pl.program_id(0)
    s = jnp.concatenate([s0_ref[...], s1_ref[...]], axis=1)
    h = x0w_ref[...] + dinv_ref[...] * s + bc_ref[...]
    h = jnp.maximum(h, 0.0)
    hp = jnp.dot(h, wg_ref[...], preferred_element_type=f32)
    hp_ref[...] = hp
    asrc = jnp.dot(hp, av_ref[...], preferred_element_type=f32)
    adst = jnp.dot(hp, ad_ref[...], preferred_element_type=f32)
    rid = i * BR + lax.broadcasted_iota(i32, (BR, 1), 0)
    asrc = jnp.where(rid < N, asrc, -1e30)
    adst = jnp.where(rid < N, adst, -1e30)
    asrc_ref[...] = asrc
    adst_ref[...] = adst
    mpart_ref[...] = jnp.full((BR, 1), jnp.max(asrc), f32)


def _t2(x0w, s0, s1, dinv, bc2, W_g, av2, ad2):
    return pl.pallas_call(
        _t2_body,
        grid=(NB,),
        in_specs=[
            pl.BlockSpec((BR, D), lambda i: (i, 0)),
            pl.BlockSpec((BR, 128), lambda i: (i, 0)),
            pl.BlockSpec((BR, 128), lambda i: (i, 0)),
            pl.BlockSpec((BR, 1), lambda i: (i, 0)),
            pl.BlockSpec((1, D), lambda i: (0, 0)),
            pl.BlockSpec((D, D), lambda i: (0, 0)),
            pl.BlockSpec((D, 1), lambda i: (0, 0)),
            pl.BlockSpec((D, 1), lambda i: (0, 0)),
        ],
        out_specs=[
            pl.BlockSpec((BR, D), lambda i: (i, 0)),
            pl.BlockSpec((BR, 1), lambda i: (i, 0)),
            pl.BlockSpec((BR, 1), lambda i: (i, 0)),
            pl.BlockSpec((BR, 1), lambda i: (i, 0)),
        ],
        out_shape=[
            jax.ShapeDtypeStruct((NP, D), f32),
            jax.ShapeDtypeStruct((NP, 1), f32),
            jax.ShapeDtypeStruct((NP, 1), f32),
            jax.ShapeDtypeStruct((NP, 1), f32),
        ],
    )(x0w, s0, s1, dinv, bc2, W_g, av2, ad2)


def _t2b_body(asrc_ref, adst_ref, m0_ref, m1_ref, emax_ref, selfw_ref):
    adst = adst_ref[...]
    asrc = asrc_ref[...]
    m = jnp.maximum(jnp.maximum(m0_ref[...], m1_ref[...]), asrc)
    v = m + adst
    emax = jnp.where(v > 0, v, 0.2 * v)
    e = asrc + adst
    e = jnp.where(e > 0, e, 0.2 * e)
    emax_ref[...] = emax
    selfw_ref[...] = e - emax


def _t2b(asrcP, adstP, m0, m1):
    return pl.pallas_call(
        _t2b_body,
        out_shape=[
            jax.ShapeDtypeStruct((NP, 1), f32),
            jax.ShapeDtypeStruct((NP, 1), f32),
        ],
    )(asrcP, adstP, m0, m1)


def _t3_body(sn0_ref, sn1_ref, hp_ref, selfw_ref, esump_ref, ones32_ref,
             batch_ref, bg_ref, wl_ref, bl_ref, out_ref):
    esum = lax.dot_general(esump_ref[...], ones32_ref[...],
                           (((0,), (0,)), ((), ())),
                           preferred_element_type=f32, precision=lax.Precision.HIGHEST)
    selfw = selfw_ref[...]
    hp = hp_ref[...]
    num = jnp.concatenate([sn0_ref[...], sn1_ref[...]], axis=1) + selfw * hp
    den = esum + selfw + 1e-16
    out = jnp.maximum(num * _recip(den) + bg_ref[...], 0.0)
    oh = (batch_ref[...] == lax.broadcasted_iota(i32, (1, G), 1)).astype(f32)
    ps = lax.dot_general(oh, out, (((0,), (0,)), ((), ())),
                         preferred_element_type=f32, precision=lax.Precision.HIGHEST)
    cnt = lax.dot_general(oh, jnp.ones((NP, 1), f32), (((0,), (0,)), ((), ())),
                          preferred_element_type=f32, precision=lax.Precision.HIGHEST)
    pooled = ps * _recip(jnp.maximum(cnt, 1.0))
    out_ref[...] = (jnp.dot(pooled, wl_ref[...], preferred_element_type=f32)
                    + bl_ref[...])


def _t3(sn0, sn1, hp, selfwP, esump, batchP, bg2, W_l, bl2):
    ones32 = jnp.ones((32, 1), f32)
    return pl.pallas_call(
        _t3_body,
        out_shape=jax.ShapeDtypeStruct((G, 1), f32),
    )(sn0, sn1, hp, selfwP, esump, ones32, batchP, bg2, W_l, bl2)


def kernel(x, edge_index, edge_attr, batch, W_c0, W_c1, b_c, W_g, a_src,
           a_dst, b_g, W_l, b_l):
    row = edge_index[0]
    col = edge_index[1]
    pad = EP - E
    rowp = jnp.concatenate([row, jnp.zeros((pad,), i32)])
    colp = jnp.concatenate([col, jnp.zeros((pad,), i32)])
    eap = jnp.concatenate([edge_attr, jnp.zeros((pad,), f32)])
    row2 = 2 * rowp
    xp = jnp.pad(x, ((0, NP - N), (0, 0)))
    batchP = jnp.concatenate([batch, jnp.full((NP - N,), G, i32)]).reshape(NP, 1)
    bc2 = b_c.reshape(1, D)
    bg2 = b_g.reshape(1, D)
    av2 = a_src.reshape(D, 1)
    ad2 = a_dst.reshape(D, 1)
    bl2 = b_l.reshape(1, 1)

    degp = _sc_deg(rowp, eap)
    x0w, y1, dinv = _t1(xp, W_c0, W_c1, degp.reshape(32, NP))
    s0, s1 = _sc_agg(y1.reshape(2 * NP, 128), row2, colp, -eap)

    hp, asrcP, adstP, mpart = _t2(x0w, s0, s1, dinv, bc2, W_g, av2, ad2)
    msrc = _sc_max(rowp, colp, asrcP.reshape(NP))
    emaxP, selfeP = _t2b(asrcP, adstP, msrc[:NP].reshape(NP, 1),
                         msrc[NP:].reshape(NP, 1))
    selfwP = jnp.exp(selfeP)
    elsh = _sc_gat(rowp, colp, asrcP.reshape(NP), adstP.reshape(NP),
                   emaxP.reshape(NP))
    wE = jnp.exp(elsh)
    esump = _sc_deg(colp, wE)
    sn0, sn1 = _sc_agg(hp.reshape(2 * NP, 128), row2, colp, wE)

    return _t3(sn0, sn1, hp, selfwP, esump.reshape(32, NP), batchP, bg2, W_l,
               bl2)
